# Initial kernel scaffold; baseline (speedup 1.0000x reference)
import jax
import jax.numpy as jnp
from jax.experimental import pallas as pl

TOKEN_BLOCK = 1024


def _moe_dense_body(x_ref, rw_ref, ri_ref, wgu_ref, wd_ref, out_ref):
    e = pl.program_id(1)
    inter = wd_ref.shape[1]

    x = x_ref[...]
    gu = jnp.dot(x, wgu_ref[0], preferred_element_type=jnp.float32)
    gate = gu[:, :inter]
    up = gu[:, inter:]
    h = (gate * jax.nn.sigmoid(gate)) * up
    contrib = jnp.dot(h.astype(jnp.bfloat16), wd_ref[0],
                      preferred_element_type=jnp.float32)

    ri = ri_ref[...]
    routed = (ri[:, 0:1] == e) | (ri[:, 1:2] == e)
    scale = jnp.where(routed, rw_ref[...][:, e][:, None], 0.0)
    contrib = contrib * scale

    @pl.when(e == 0)
    def _init():
        out_ref[...] = contrib

    @pl.when(e > 0)
    def _acc():
        out_ref[...] += contrib


def kernel(hidden_states, routing_weights, router_indices, gate_up_proj,
           down_proj):
    b, s, hidden = hidden_states.shape
    n_tok = b * s
    num_experts, _, two_inter = gate_up_proj.shape
    inter = two_inter // 2

    hs = hidden_states.reshape(n_tok, hidden).astype(jnp.bfloat16)
    rw = routing_weights.reshape(n_tok, num_experts)
    ri = router_indices.reshape(n_tok, -1).astype(jnp.int32)
    wgu = gate_up_proj.astype(jnp.bfloat16)
    wd = down_proj.astype(jnp.bfloat16)

    nt = n_tok // TOKEN_BLOCK
    grid = (nt, num_experts)

    out = pl.pallas_call(
        _moe_dense_body,
        grid=grid,
        in_specs=[
            pl.BlockSpec((TOKEN_BLOCK, hidden), lambda i, e: (i, 0)),
            pl.BlockSpec((TOKEN_BLOCK, num_experts), lambda i, e: (i, 0)),
            pl.BlockSpec((TOKEN_BLOCK, 2), lambda i, e: (i, 0)),
            pl.BlockSpec((1, hidden, two_inter), lambda i, e: (e, 0, 0)),
            pl.BlockSpec((1, inter, hidden), lambda i, e: (e, 0, 0)),
        ],
        out_specs=pl.BlockSpec((TOKEN_BLOCK, hidden), lambda i, e: (i, 0)),
        out_shape=jax.ShapeDtypeStruct((n_tok, hidden), jnp.float32),
    )(hs, rw, ri, wgu, wd)

    return out.reshape(b, s, hidden)


# speedup vs baseline: 1.2800x; 1.2800x over previous
import jax
import jax.numpy as jnp
from jax.experimental import pallas as pl

TOKEN_BLOCK = 1024


def _moe_dense_body(x_ref, rw_ref, ri_ref, wgu_ref, wd_ref, out_ref):
    e = pl.program_id(1)
    inter = wd_ref.shape[1]

    x = x_ref[...]
    gu = jnp.dot(x, wgu_ref[0], preferred_element_type=jnp.float32)
    gate = gu[:, :inter]
    up = gu[:, inter:]
    h = (gate * jax.nn.sigmoid(gate)) * up
    contrib = jnp.dot(h.astype(jnp.bfloat16), wd_ref[0],
                      preferred_element_type=jnp.float32)

    ri = ri_ref[...]
    routed = (ri[:, 0:1] == e) | (ri[:, 1:2] == e)
    rw = rw_ref[...]
    col = jax.lax.broadcasted_iota(jnp.int32, rw.shape, 1)
    w_e = jnp.sum(jnp.where(col == e, rw, 0.0), axis=1, keepdims=True)
    contrib = contrib * jnp.where(routed, w_e, 0.0)

    @pl.when(e == 0)
    def _init():
        out_ref[...] = contrib

    @pl.when(e > 0)
    def _acc():
        out_ref[...] += contrib


def kernel(hidden_states, routing_weights, router_indices, gate_up_proj,
           down_proj):
    b, s, hidden = hidden_states.shape
    n_tok = b * s
    num_experts, _, two_inter = gate_up_proj.shape
    inter = two_inter // 2

    hs = hidden_states.reshape(n_tok, hidden).astype(jnp.bfloat16)
    rw = routing_weights.reshape(n_tok, num_experts)
    ri = router_indices.reshape(n_tok, -1).astype(jnp.int32)
    wgu = gate_up_proj.astype(jnp.bfloat16)
    wd = down_proj.astype(jnp.bfloat16)

    nt = n_tok // TOKEN_BLOCK
    grid = (nt, num_experts)

    out = pl.pallas_call(
        _moe_dense_body,
        grid=grid,
        in_specs=[
            pl.BlockSpec((TOKEN_BLOCK, hidden), lambda i, e: (i, 0)),
            pl.BlockSpec((TOKEN_BLOCK, num_experts), lambda i, e: (i, 0)),
            pl.BlockSpec((TOKEN_BLOCK, 2), lambda i, e: (i, 0)),
            pl.BlockSpec((1, hidden, two_inter), lambda i, e: (e, 0, 0)),
            pl.BlockSpec((1, inter, hidden), lambda i, e: (e, 0, 0)),
        ],
        out_specs=pl.BlockSpec((TOKEN_BLOCK, hidden), lambda i, e: (i, 0)),
        out_shape=jax.ShapeDtypeStruct((n_tok, hidden), jnp.float32),
    )(hs, rw, ri, wgu, wd)

    return out.reshape(b, s, hidden)
